# baseline (speedup 1.0000x reference)
import functools

import jax
import jax.numpy as jnp
from jax import lax
from jax.experimental import pallas as pl
from jax.experimental.pallas import tpu as pltpu
from jax.experimental.pallas import tpu_sc as plsc

_B, _N = 64, 8192
_K = 820
_L = 16
_NC, _NS = 2, 16
_NW = _NC * _NS
_RPW = _B // _NW
_NV = _N // _L
_U = 8
_H = 4
_MIN32 = -2**31
_NINF_BITS = -8388608

_mesh = plsc.VectorSubcoreMesh(core_axis_name="c", subcore_axis_name="s")


@functools.partial(
    pl.kernel,
    out_type=jax.ShapeDtypeStruct((_B, _N), jnp.int32),
    mesh=_mesh,
    compiler_params=pltpu.CompilerParams(needs_layout_passes=False),
    scratch_types=[
        pltpu.VMEM((_N,), jnp.int32),
        pltpu.VMEM((_N,), jnp.int32),
        pltpu.VMEM((_N,), jnp.int32),
        pltpu.VMEM((_N,), jnp.int32),
        pltpu.VMEM((_H * 256,), jnp.int32),
        pltpu.SemaphoreType.DMA,
        pltpu.SemaphoreType.DMA,
        pltpu.SemaphoreType.DMA,
        pltpu.SemaphoreType.DMA,
    ],
)
def _sc_topk_mask(bits_hbm, out_hbm, xa, xb, oa, ob,
                  hist, s_ia, s_ib, s_oa, s_ob):
    wid = lax.axis_index("s") * _NC + lax.axis_index("c")
    iota = lax.iota(jnp.int32, _L)
    ones = jnp.ones((_L,), jnp.int32)
    zeros16 = jnp.zeros((_L,), jnp.int32)
    ninfv = jnp.full((_L,), _NINF_BITS, jnp.int32)

    def process(xi, out):
        c_gt = jnp.int32(0)
        prefix = jnp.int32(0)
        hb = jnp.int32(0)
        for p in range(4):
            shift = 24 - 8 * p
            pmask = (1 << (8 * p)) - 1

            for c in range(_H * _L):
                hist[pl.ds(c * _L, _L)] = zeros16

            def sbody(j, _, _p=p, _shift=shift, _pm=pmask, _prefix=prefix):
                for u in range(_U):
                    sl = pl.ds((j * _U + u) * _L, _L)
                    hoff = (u % _H) * 256
                    if _p == 0:
                        iv = xi[sl]
                        ub = jnp.where(iv < 0, ~iv, iv ^ _MIN32)
                        xi[sl] = ub
                        binv = ((ub >> _shift) & 0xFF) + hoff
                        plsc.addupdate_scatter(hist, [binv], ones)
                    else:
                        ub = xi[sl]
                        binv = ((ub >> _shift) & 0xFF) + hoff
                        sel = ((ub >> (_shift + 8)) & _pm) == _prefix
                        plsc.addupdate_scatter(hist, [binv], ones, mask=sel)
                return 0

            with jax.named_scope(f"scatter{p}"):
                lax.fori_loop(0, _NV // _U, sbody, 0)

            with jax.named_scope(f"select{p}"):
                csv = zeros16
                for c in range(_L):
                    acc = hist[pl.ds(c * _L, _L)]
                    for h in range(1, _H):
                        acc = acc + hist[pl.ds(h * 256 + c * _L, _L)]
                    s = jnp.sum(acc)
                    csv = jnp.where(iota == c, s, csv)
                suff_c = jnp.flip(jnp.cumsum(jnp.flip(csv)))
                cstar = jnp.max(jnp.where((c_gt + suff_c) >= _K, iota, -1))
                rnext = jnp.max(jnp.where(iota == cstar, suff_c - csv, -1))

                w = hist[pl.ds(cstar * _L, _L)]
                for h in range(1, _H):
                    w = w + hist[pl.ds(h * 256 + cstar * _L, _L)]
                suff_w = jnp.flip(jnp.cumsum(jnp.flip(w)))
                bsel = jnp.max(
                    jnp.where((c_gt + rnext + suff_w) >= _K, iota, -1))
                hb = jnp.max(jnp.where(iota == bsel, w, -1))
                suffb = jnp.max(jnp.where(iota == bsel, suff_w, -1))
                c_gt = c_gt + rnext + suffb - hb
                prefix = (prefix << 8) | (cstar * _L + bsel)

        tkey = prefix ^ _MIN32

        def mbody(j, _):
            for u in range(_U):
                sl = pl.ds((j * _U + u) * _L, _L)
                ub = xi[sl]
                ks = ub ^ _MIN32
                iv = jnp.where(ks < 0, ~ub, ks)
                out[sl] = jnp.where(ks >= tkey, iv, ninfv)
            return 0

        with jax.named_scope("maskpass"):
            lax.fori_loop(0, _NV // _U, mbody, 0)

        quota = jnp.int32(_K) - c_gt

        @pl.when(c_gt + hb > _K)
        def _():
            def fbody(j, cnt):
                sl = pl.ds(j * _L, _L)
                eqm = xi[sl] == prefix
                incl = jnp.cumsum(eqm.astype(jnp.int32))
                kill = eqm & ((cnt + incl) > quota)
                out[sl] = jnp.where(kill, ninfv, out[sl])
                return cnt + plsc.all_reduce_population_count(eqm)

            lax.fori_loop(0, _NV, fbody, jnp.zeros((_L,), jnp.int32))

    row = wid * _RPW
    cp_ia = pltpu.async_copy(bits_hbm.at[row], xa, s_ia)
    cp_ib = pltpu.async_copy(bits_hbm.at[row + 1], xb, s_ib)
    cp_ia.wait()
    process(xa, oa)
    cp_oa = pltpu.async_copy(oa, out_hbm.at[row], s_oa)
    cp_ib.wait()
    process(xb, ob)
    cp_ob = pltpu.async_copy(ob, out_hbm.at[row + 1], s_ob)
    cp_oa.wait()
    cp_ob.wait()


def kernel(scores, k):
    bits = lax.bitcast_convert_type(scores, jnp.int32)
    out = lax.bitcast_convert_type(_sc_topk_mask(bits), jnp.float32)
    return out + (k * 0)


# speedup vs baseline: 1.6956x; 1.6956x over previous
import functools

import jax
import jax.numpy as jnp
from jax import lax
from jax.experimental import pallas as pl
from jax.experimental.pallas import tpu as pltpu
from jax.experimental.pallas import tpu_sc as plsc

_B, _N = 64, 8192
_K = 820
_L = 16
_NC, _NS = 2, 16
_NW = _NC * _NS
_RPW = _B // _NW
_NV = _N // _L
_U = 8
_H = 4
_MIN32 = -2**31
_NINF_BITS = -8388608

_mesh = plsc.VectorSubcoreMesh(core_axis_name="c", subcore_axis_name="s")


@functools.partial(
    pl.kernel,
    out_type=jax.ShapeDtypeStruct((_B, _N), jnp.int32),
    mesh=_mesh,
    compiler_params=pltpu.CompilerParams(needs_layout_passes=False),
    scratch_types=[
        pltpu.VMEM((_N,), jnp.int32),
        pltpu.VMEM((_N,), jnp.int32),
        pltpu.VMEM((_N,), jnp.int32),
        pltpu.VMEM((_N,), jnp.int32),
        pltpu.VMEM((_H * 256,), jnp.int32),
        pltpu.SemaphoreType.DMA,
        pltpu.SemaphoreType.DMA,
        pltpu.SemaphoreType.DMA,
        pltpu.SemaphoreType.DMA,
    ],
)
def _sc_topk_mask(bits_hbm, out_hbm, xa, xb, oa, ob,
                  hist, s_ia, s_ib, s_oa, s_ob):
    wid = lax.axis_index("s") * _NC + lax.axis_index("c")
    iota = lax.iota(jnp.int32, _L)
    ones = jnp.ones((_L,), jnp.int32)
    zeros16 = jnp.zeros((_L,), jnp.int32)
    ninfv = jnp.full((_L,), _NINF_BITS, jnp.int32)

    def process(xi, out):
        c_gt = jnp.int32(0)
        prefix = jnp.int32(0)
        hb = jnp.int32(0)
        for p in range(4):
            shift = 24 - 8 * p
            pmask = (1 << (8 * p)) - 1

            for c in range(_H * _L):
                hist[pl.ds(c * _L, _L)] = zeros16

            with jax.named_scope(f"scatter{p}"):
                @plsc.parallel_loop(0, _NV, unroll=_U)
                def _(j, _p=p, _shift=shift, _pm=pmask, _prefix=prefix):
                    sl = pl.ds(j * _L, _L)
                    hoff = (j % _H) * 256
                    if _p == 0:
                        iv = xi[sl]
                        ub = jnp.where(iv < 0, ~iv, iv ^ _MIN32)
                        xi[sl] = ub
                        binv = ((ub >> _shift) & 0xFF) + hoff
                        plsc.addupdate_scatter(hist, [binv], ones)
                    else:
                        ub = xi[sl]
                        binv = ((ub >> _shift) & 0xFF) + hoff
                        sel = ((ub >> (_shift + 8)) & _pm) == _prefix
                        plsc.addupdate_scatter(hist, [binv], ones, mask=sel)

            with jax.named_scope(f"select{p}"):
                csv = zeros16
                for c in range(_L):
                    acc = hist[pl.ds(c * _L, _L)]
                    for h in range(1, _H):
                        acc = acc + hist[pl.ds(h * 256 + c * _L, _L)]
                    s = jnp.sum(acc)
                    csv = jnp.where(iota == c, s, csv)
                suff_c = jnp.flip(jnp.cumsum(jnp.flip(csv)))
                cstar = jnp.max(jnp.where((c_gt + suff_c) >= _K, iota, -1))
                rnext = jnp.max(jnp.where(iota == cstar, suff_c - csv, -1))

                w = hist[pl.ds(cstar * _L, _L)]
                for h in range(1, _H):
                    w = w + hist[pl.ds(h * 256 + cstar * _L, _L)]
                suff_w = jnp.flip(jnp.cumsum(jnp.flip(w)))
                bsel = jnp.max(
                    jnp.where((c_gt + rnext + suff_w) >= _K, iota, -1))
                hb = jnp.max(jnp.where(iota == bsel, w, -1))
                suffb = jnp.max(jnp.where(iota == bsel, suff_w, -1))
                c_gt = c_gt + rnext + suffb - hb
                prefix = (prefix << 8) | (cstar * _L + bsel)

        tkey = prefix ^ _MIN32

        with jax.named_scope("maskpass"):
            @plsc.parallel_loop(0, _NV, unroll=_U)
            def _(j):
                sl = pl.ds(j * _L, _L)
                ub = xi[sl]
                ks = ub ^ _MIN32
                iv = jnp.where(ks < 0, ~ub, ks)
                out[sl] = jnp.where(ks >= tkey, iv, ninfv)

        quota = jnp.int32(_K) - c_gt

        @pl.when(c_gt + hb > _K)
        def _():
            def fbody(j, cnt):
                sl = pl.ds(j * _L, _L)
                eqm = xi[sl] == prefix
                incl = jnp.cumsum(eqm.astype(jnp.int32))
                kill = eqm & ((cnt + incl) > quota)
                out[sl] = jnp.where(kill, ninfv, out[sl])
                return cnt + plsc.all_reduce_population_count(eqm)

            lax.fori_loop(0, _NV, fbody, jnp.zeros((_L,), jnp.int32))

    row = wid * _RPW
    cp_ia = pltpu.async_copy(bits_hbm.at[row], xa, s_ia)
    cp_ib = pltpu.async_copy(bits_hbm.at[row + 1], xb, s_ib)
    cp_ia.wait()
    process(xa, oa)
    cp_oa = pltpu.async_copy(oa, out_hbm.at[row], s_oa)
    cp_ib.wait()
    process(xb, ob)
    cp_ob = pltpu.async_copy(ob, out_hbm.at[row + 1], s_ob)
    cp_oa.wait()
    cp_ob.wait()


def kernel(scores, k):
    bits = lax.bitcast_convert_type(scores, jnp.int32)
    out = lax.bitcast_convert_type(_sc_topk_mask(bits), jnp.float32)
    return out + (k * 0)


# speedup vs baseline: 1.8047x; 1.0643x over previous
import functools

import jax
import jax.numpy as jnp
from jax import lax
from jax.experimental import pallas as pl
from jax.experimental.pallas import tpu as pltpu
from jax.experimental.pallas import tpu_sc as plsc

_B, _N = 64, 8192
_K = 820
_L = 16
_NC, _NS = 2, 16
_NW = _NC * _NS
_RPW = _B // _NW
_NV = _N // _L
_U = 8
_H = 4
_MIN32 = -2**31
_NINF_BITS = -8388608

_mesh = plsc.VectorSubcoreMesh(core_axis_name="c", subcore_axis_name="s")


@functools.partial(
    pl.kernel,
    out_type=jax.ShapeDtypeStruct((_B, _N), jnp.float32),
    mesh=_mesh,
    compiler_params=pltpu.CompilerParams(needs_layout_passes=False),
    scratch_types=[
        pltpu.VMEM((_N,), jnp.int32),
        pltpu.VMEM((_N,), jnp.int32),
        pltpu.VMEM((_N,), jnp.float32),
        pltpu.VMEM((_N,), jnp.float32),
        pltpu.VMEM((_N,), jnp.float32),
        pltpu.VMEM((_N,), jnp.float32),
        pltpu.VMEM((_H * 256,), jnp.int32),
        pltpu.SemaphoreType.DMA,
        pltpu.SemaphoreType.DMA,
        pltpu.SemaphoreType.DMA,
        pltpu.SemaphoreType.DMA,
        pltpu.SemaphoreType.DMA,
        pltpu.SemaphoreType.DMA,
    ],
)
def _sc_topk_mask(scores_hbm, bits_hbm, out_hbm, xa, xb, fa, fb, oa, ob,
                  hist, s_ia, s_ib, s_fa, s_fb, s_oa, s_ob):
    wid = lax.axis_index("s") * _NC + lax.axis_index("c")
    iota = lax.iota(jnp.int32, _L)
    ones = jnp.ones((_L,), jnp.int32)
    zeros16 = jnp.zeros((_L,), jnp.int32)
    ninfv = jnp.full((_L,), -jnp.inf, jnp.float32)

    def process(xi, xf, out):
        c_gt = jnp.int32(0)
        prefix = jnp.int32(0)
        hb = jnp.int32(0)
        for p in range(4):
            shift = 24 - 8 * p
            pmask = (1 << (8 * p)) - 1

            for c in range(_H * _L):
                hist[pl.ds(c * _L, _L)] = zeros16

            with jax.named_scope(f"scatter{p}"):
                @plsc.parallel_loop(0, _NV, unroll=_U)
                def _(j, _p=p, _shift=shift, _pm=pmask, _prefix=prefix):
                    sl = pl.ds(j * _L, _L)
                    hoff = (j % _H) * 256
                    if _p == 0:
                        iv = xi[sl]
                        ub = jnp.where(iv < 0, ~iv, iv ^ _MIN32)
                        xi[sl] = ub
                        binv = ((ub >> _shift) & 0xFF) + hoff
                        plsc.addupdate_scatter(hist, [binv], ones)
                    else:
                        ub = xi[sl]
                        binv = ((ub >> _shift) & 0xFF) + hoff
                        sel = ((ub >> (_shift + 8)) & _pm) == _prefix
                        plsc.addupdate_scatter(hist, [binv], ones, mask=sel)

            with jax.named_scope(f"select{p}"):
                csv = zeros16
                for c in range(_L):
                    acc = hist[pl.ds(c * _L, _L)]
                    for h in range(1, _H):
                        acc = acc + hist[pl.ds(h * 256 + c * _L, _L)]
                    s = jnp.sum(acc)
                    csv = jnp.where(iota == c, s, csv)
                suff_c = jnp.flip(jnp.cumsum(jnp.flip(csv)))
                cstar = jnp.max(jnp.where((c_gt + suff_c) >= _K, iota, -1))
                rnext = jnp.max(jnp.where(iota == cstar, suff_c - csv, -1))

                w = hist[pl.ds(cstar * _L, _L)]
                for h in range(1, _H):
                    w = w + hist[pl.ds(h * 256 + cstar * _L, _L)]
                suff_w = jnp.flip(jnp.cumsum(jnp.flip(w)))
                bsel = jnp.max(
                    jnp.where((c_gt + rnext + suff_w) >= _K, iota, -1))
                hb = jnp.max(jnp.where(iota == bsel, w, -1))
                suffb = jnp.max(jnp.where(iota == bsel, suff_w, -1))
                c_gt = c_gt + rnext + suffb - hb
                prefix = (prefix << 8) | (cstar * _L + bsel)

        tkey = prefix ^ _MIN32

        with jax.named_scope("maskpass"):
            @plsc.parallel_loop(0, _NV, unroll=_U)
            def _(j):
                sl = pl.ds(j * _L, _L)
                ks = xi[sl] ^ _MIN32
                out[sl] = jnp.where(ks >= tkey, xf[sl], ninfv)

        quota = jnp.int32(_K) - c_gt

        @pl.when(c_gt + hb > _K)
        def _():
            def fbody(j, cnt):
                sl = pl.ds(j * _L, _L)
                eqm = xi[sl] == prefix
                incl = jnp.cumsum(eqm.astype(jnp.int32))
                kill = eqm & ((cnt + incl) > quota)
                out[sl] = jnp.where(kill, ninfv, out[sl])
                return cnt + plsc.all_reduce_population_count(eqm)

            lax.fori_loop(0, _NV, fbody, jnp.zeros((_L,), jnp.int32))

    row = wid * _RPW
    cp_ia = pltpu.async_copy(bits_hbm.at[row], xa, s_ia)
    cp_fa = pltpu.async_copy(scores_hbm.at[row], fa, s_fa)
    cp_ib = pltpu.async_copy(bits_hbm.at[row + 1], xb, s_ib)
    cp_fb = pltpu.async_copy(scores_hbm.at[row + 1], fb, s_fb)
    cp_ia.wait()
    cp_fa.wait()
    process(xa, fa, oa)
    cp_oa = pltpu.async_copy(oa, out_hbm.at[row], s_oa)
    cp_ib.wait()
    cp_fb.wait()
    process(xb, fb, ob)
    cp_ob = pltpu.async_copy(ob, out_hbm.at[row + 1], s_ob)
    cp_oa.wait()
    cp_ob.wait()


def kernel(scores, k):
    bits = lax.bitcast_convert_type(scores, jnp.int32)
    out = _sc_topk_mask(scores, bits)
    return out + (k * 0)


# speedup vs baseline: 1.8108x; 1.0034x over previous
import functools

import jax
import jax.numpy as jnp
from jax import lax
from jax.experimental import pallas as pl
from jax.experimental.pallas import tpu as pltpu
from jax.experimental.pallas import tpu_sc as plsc

_B, _N = 64, 8192
_K = 820
_L = 16
_NC, _NS = 2, 16
_NW = _NC * _NS
_RPW = _B // _NW
_NV = _N // _L
_U = 8
_H = 4
_MIN32 = -2**31
_NINF_BITS = -8388608

_mesh = plsc.VectorSubcoreMesh(core_axis_name="c", subcore_axis_name="s")


@functools.partial(
    pl.kernel,
    out_type=jax.ShapeDtypeStruct((_B, _N), jnp.float32),
    mesh=_mesh,
    compiler_params=pltpu.CompilerParams(needs_layout_passes=False),
    scratch_types=[
        pltpu.VMEM((_N,), jnp.int32),
        pltpu.VMEM((_N,), jnp.int32),
        pltpu.VMEM((_N,), jnp.float32),
        pltpu.VMEM((_N,), jnp.float32),
        pltpu.VMEM((_N,), jnp.float32),
        pltpu.VMEM((_N,), jnp.float32),
        pltpu.VMEM((_H * 256,), jnp.int32),
        pltpu.SemaphoreType.DMA,
        pltpu.SemaphoreType.DMA,
        pltpu.SemaphoreType.DMA,
        pltpu.SemaphoreType.DMA,
        pltpu.SemaphoreType.DMA,
        pltpu.SemaphoreType.DMA,
    ],
)
def _sc_topk_mask(scores_hbm, out_hbm, xa, xb, fa, fb, oa, ob,
                  hist, s_ia, s_ib, s_fa, s_fb, s_oa, s_ob):
    wid = lax.axis_index("s") * _NC + lax.axis_index("c")
    iota = lax.iota(jnp.int32, _L)
    ones = jnp.ones((_L,), jnp.int32)
    zeros16 = jnp.zeros((_L,), jnp.int32)
    ninfv = jnp.full((_L,), -jnp.inf, jnp.float32)

    def process(xi, xf, out):
        c_gt = jnp.int32(0)
        prefix = jnp.int32(0)
        hb = jnp.int32(0)
        for p in range(4):
            shift = 24 - 8 * p
            pmask = (1 << (8 * p)) - 1

            for c in range(_H * _L):
                hist[pl.ds(c * _L, _L)] = zeros16

            with jax.named_scope(f"scatter{p}"):
                @plsc.parallel_loop(0, _NV, unroll=_U)
                def _(j, _p=p, _shift=shift, _pm=pmask, _prefix=prefix):
                    sl = pl.ds(j * _L, _L)
                    hoff = (j % _H) * 256
                    if _p == 0:
                        iv = xi[sl]
                        ub = jnp.where(iv < 0, ~iv, iv ^ _MIN32)
                        xi[sl] = ub
                        binv = ((ub >> _shift) & 0xFF) + hoff
                        plsc.addupdate_scatter(hist, [binv], ones)
                    else:
                        ub = xi[sl]
                        binv = ((ub >> _shift) & 0xFF) + hoff
                        sel = ((ub >> (_shift + 8)) & _pm) == _prefix
                        plsc.addupdate_scatter(hist, [binv], ones, mask=sel)

            with jax.named_scope(f"select{p}"):
                csv = zeros16
                for c in range(_L):
                    acc = hist[pl.ds(c * _L, _L)]
                    for h in range(1, _H):
                        acc = acc + hist[pl.ds(h * 256 + c * _L, _L)]
                    s = jnp.sum(acc)
                    csv = jnp.where(iota == c, s, csv)
                suff_c = jnp.flip(jnp.cumsum(jnp.flip(csv)))
                cstar = jnp.max(jnp.where((c_gt + suff_c) >= _K, iota, -1))
                rnext = jnp.max(jnp.where(iota == cstar, suff_c - csv, -1))

                w = hist[pl.ds(cstar * _L, _L)]
                for h in range(1, _H):
                    w = w + hist[pl.ds(h * 256 + cstar * _L, _L)]
                suff_w = jnp.flip(jnp.cumsum(jnp.flip(w)))
                bsel = jnp.max(
                    jnp.where((c_gt + rnext + suff_w) >= _K, iota, -1))
                hb = jnp.max(jnp.where(iota == bsel, w, -1))
                suffb = jnp.max(jnp.where(iota == bsel, suff_w, -1))
                c_gt = c_gt + rnext + suffb - hb
                prefix = (prefix << 8) | (cstar * _L + bsel)

        tkey = prefix ^ _MIN32

        with jax.named_scope("maskpass"):
            @plsc.parallel_loop(0, _NV, unroll=_U)
            def _(j):
                sl = pl.ds(j * _L, _L)
                ks = xi[sl] ^ _MIN32
                out[sl] = jnp.where(ks >= tkey, xf[sl], ninfv)

        quota = jnp.int32(_K) - c_gt

        @pl.when(c_gt + hb > _K)
        def _():
            def fbody(j, cnt):
                sl = pl.ds(j * _L, _L)
                eqm = xi[sl] == prefix
                incl = jnp.cumsum(eqm.astype(jnp.int32))
                kill = eqm & ((cnt + incl) > quota)
                out[sl] = jnp.where(kill, ninfv, out[sl])
                return cnt + plsc.all_reduce_population_count(eqm)

            lax.fori_loop(0, _NV, fbody, jnp.zeros((_L,), jnp.int32))

    row = wid * _RPW
    bits_hbm = scores_hbm.at[...].bitcast(jnp.int32)
    cp_ia = pltpu.async_copy(bits_hbm.at[row], xa, s_ia)
    cp_fa = pltpu.async_copy(scores_hbm.at[row], fa, s_fa)
    cp_ib = pltpu.async_copy(bits_hbm.at[row + 1], xb, s_ib)
    cp_fb = pltpu.async_copy(scores_hbm.at[row + 1], fb, s_fb)
    cp_ia.wait()
    cp_fa.wait()
    process(xa, fa, oa)
    cp_oa = pltpu.async_copy(oa, out_hbm.at[row], s_oa)
    cp_ib.wait()
    cp_fb.wait()
    process(xb, fb, ob)
    cp_ob = pltpu.async_copy(ob, out_hbm.at[row + 1], s_ob)
    cp_oa.wait()
    cp_ob.wait()


def kernel(scores, k):
    out = _sc_topk_mask(scores)
    return out + (k * 0)


# speedup vs baseline: 1.9380x; 1.0703x over previous
import functools

import jax
import jax.numpy as jnp
from jax import lax
from jax.experimental import pallas as pl
from jax.experimental.pallas import tpu as pltpu
from jax.experimental.pallas import tpu_sc as plsc

_B, _N = 64, 8192
_K = 820
_L = 16
_NC, _NS = 2, 16
_NW = _NC * _NS
_RPW = _B // _NW
_NV = _N // _L
_U = 8
_H = 4
_MIN32 = -2**31
_NINF_BITS = -8388608

_mesh = plsc.VectorSubcoreMesh(core_axis_name="c", subcore_axis_name="s")


@functools.partial(
    pl.kernel,
    out_type=jax.ShapeDtypeStruct((_B, _N), jnp.float32),
    mesh=_mesh,
    compiler_params=pltpu.CompilerParams(needs_layout_passes=False),
    scratch_types=[
        pltpu.VMEM((_N,), jnp.int32),
        pltpu.VMEM((_N,), jnp.int32),
        pltpu.VMEM((_N,), jnp.float32),
        pltpu.VMEM((_N,), jnp.float32),
        pltpu.VMEM((_N,), jnp.float32),
        pltpu.VMEM((_N,), jnp.float32),
        pltpu.VMEM((256 * _L,), jnp.int32),
        pltpu.VMEM((_H * 256,), jnp.int32),
        pltpu.SemaphoreType.DMA,
        pltpu.SemaphoreType.DMA,
        pltpu.SemaphoreType.DMA,
        pltpu.SemaphoreType.DMA,
        pltpu.SemaphoreType.DMA,
        pltpu.SemaphoreType.DMA,
    ],
)
def _sc_topk_mask(scores_hbm, out_hbm, xa, xb, fa, fb, oa, ob,
                  hist0, hist, s_ia, s_ib, s_fa, s_fb, s_oa, s_ob):
    wid = lax.axis_index("s") * _NC + lax.axis_index("c")
    iota = lax.iota(jnp.int32, _L)
    ones = jnp.ones((_L,), jnp.int32)
    zeros16 = jnp.zeros((_L,), jnp.int32)
    ninfv = jnp.full((_L,), -jnp.inf, jnp.float32)

    def process(xi, xf, out):
        c_gt = jnp.int32(0)
        prefix = jnp.int32(0)
        hb = jnp.int32(0)
        for p in range(4):
            shift = 24 - 8 * p
            pmask = (1 << (8 * p)) - 1

            if p == 0:
                @plsc.parallel_loop(0, 256, unroll=8)
                def _(i):
                    hist0[pl.ds(i * _L, _L)] = zeros16
            else:
                for c in range(_H * _L):
                    hist[pl.ds(c * _L, _L)] = zeros16

            with jax.named_scope(f"scatter{p}"):
                @plsc.parallel_loop(0, _NV, unroll=_U)
                def _(j, _p=p, _shift=shift, _pm=pmask, _prefix=prefix):
                    sl = pl.ds(j * _L, _L)
                    if _p == 0:
                        iv = xi[sl]
                        ub = jnp.where(iv < 0, ~iv, iv ^ _MIN32)
                        xi[sl] = ub
                        idx = (((ub >> _shift) & 0xFF) * _L) + iota
                        plsc.addupdate_scatter(hist0, [idx], ones)
                    else:
                        ub = xi[sl]
                        binv = ((ub >> _shift) & 0xFF) + (j % _H) * 256
                        sel = ((ub >> (_shift + 8)) & _pm) == _prefix
                        plsc.addupdate_scatter(hist, [binv], ones, mask=sel)

            with jax.named_scope(f"select{p}"):
                if p == 0:
                    def cbody0(c, csv):
                        vs = [hist0[pl.ds(c * 256 + t * _L, _L)]
                              for t in range(_L)]
                        while len(vs) > 1:
                            vs = [a + b for a, b in zip(vs[::2], vs[1::2])]
                        return jnp.where(iota == c, jnp.sum(vs[0]), csv)

                    csv = lax.fori_loop(0, _L, cbody0, zeros16)
                else:
                    csv = zeros16
                    for c in range(_L):
                        acc = hist[pl.ds(c * _L, _L)]
                        for h in range(1, _H):
                            acc = acc + hist[pl.ds(h * 256 + c * _L, _L)]
                        csv = jnp.where(iota == c, jnp.sum(acc), csv)
                suff_c = jnp.flip(jnp.cumsum(jnp.flip(csv)))
                cstar = jnp.max(jnp.where((c_gt + suff_c) >= _K, iota, -1))
                rnext = jnp.max(jnp.where(iota == cstar, suff_c - csv, -1))

                if p == 0:
                    def bbody0(l, w):
                        s = jnp.sum(hist0[pl.ds(cstar * 256 + l * _L, _L)])
                        return jnp.where(iota == l, s, w)

                    w = lax.fori_loop(0, _L, bbody0, zeros16)
                else:
                    w = hist[pl.ds(cstar * _L, _L)]
                    for h in range(1, _H):
                        w = w + hist[pl.ds(h * 256 + cstar * _L, _L)]
                suff_w = jnp.flip(jnp.cumsum(jnp.flip(w)))
                bsel = jnp.max(
                    jnp.where((c_gt + rnext + suff_w) >= _K, iota, -1))
                hb = jnp.max(jnp.where(iota == bsel, w, -1))
                suffb = jnp.max(jnp.where(iota == bsel, suff_w, -1))
                c_gt = c_gt + rnext + suffb - hb
                prefix = (prefix << 8) | (cstar * _L + bsel)

        tkey = prefix ^ _MIN32

        with jax.named_scope("maskpass"):
            @plsc.parallel_loop(0, _NV, unroll=_U)
            def _(j):
                sl = pl.ds(j * _L, _L)
                ks = xi[sl] ^ _MIN32
                out[sl] = jnp.where(ks >= tkey, xf[sl], ninfv)

        quota = jnp.int32(_K) - c_gt

        @pl.when(c_gt + hb > _K)
        def _():
            def fbody(j, cnt):
                sl = pl.ds(j * _L, _L)
                eqm = xi[sl] == prefix
                incl = jnp.cumsum(eqm.astype(jnp.int32))
                kill = eqm & ((cnt + incl) > quota)
                out[sl] = jnp.where(kill, ninfv, out[sl])
                return cnt + plsc.all_reduce_population_count(eqm)

            lax.fori_loop(0, _NV, fbody, jnp.zeros((_L,), jnp.int32))

    row = wid * _RPW
    bits_hbm = scores_hbm.at[...].bitcast(jnp.int32)
    cp_ia = pltpu.async_copy(bits_hbm.at[row], xa, s_ia)
    cp_fa = pltpu.async_copy(scores_hbm.at[row], fa, s_fa)
    cp_ib = pltpu.async_copy(bits_hbm.at[row + 1], xb, s_ib)
    cp_fb = pltpu.async_copy(scores_hbm.at[row + 1], fb, s_fb)
    cp_ia.wait()
    cp_fa.wait()
    process(xa, fa, oa)
    cp_oa = pltpu.async_copy(oa, out_hbm.at[row], s_oa)
    cp_ib.wait()
    cp_fb.wait()
    process(xb, fb, ob)
    cp_ob = pltpu.async_copy(ob, out_hbm.at[row + 1], s_ob)
    cp_oa.wait()
    cp_ob.wait()


def kernel(scores, k):
    del k
    return _sc_topk_mask(scores)


# speedup vs baseline: 2.0317x; 1.0483x over previous
import functools

import jax
import jax.numpy as jnp
from jax import lax
from jax.experimental import pallas as pl
from jax.experimental.pallas import tpu as pltpu
from jax.experimental.pallas import tpu_sc as plsc

_B, _N = 64, 8192
_K = 820
_L = 16
_NC, _NS = 2, 16
_NW = _NC * _NS
_RPW = _B // _NW
_NV = _N // _L
_U = 8
_H = 4
_MIN32 = -2**31
_NINF_BITS = -8388608

_mesh = plsc.VectorSubcoreMesh(core_axis_name="c", subcore_axis_name="s")


@functools.partial(
    pl.kernel,
    out_type=jax.ShapeDtypeStruct((_B, _N), jnp.float32),
    mesh=_mesh,
    compiler_params=pltpu.CompilerParams(needs_layout_passes=False),
    scratch_types=[
        pltpu.VMEM((_N,), jnp.int32),
        pltpu.VMEM((_N,), jnp.int32),
        pltpu.VMEM((_N,), jnp.int32),
        pltpu.VMEM((_N,), jnp.int32),
        pltpu.VMEM((256 * _L,), jnp.int32),
        pltpu.VMEM((_H * 256,), jnp.int32),
        pltpu.SemaphoreType.DMA,
        pltpu.SemaphoreType.DMA,
        pltpu.SemaphoreType.DMA,
        pltpu.SemaphoreType.DMA,
    ],
)
def _sc_topk_mask(scores_hbm, out_hbm, xa, xb, oa, ob,
                  hist0, hist, s_ia, s_ib, s_oa, s_ob):
    wid = lax.axis_index("s") * _NC + lax.axis_index("c")
    iota = lax.iota(jnp.int32, _L)
    ones = jnp.ones((_L,), jnp.int32)
    zeros16 = jnp.zeros((_L,), jnp.int32)
    ninfv = jnp.full((_L,), _NINF_BITS, jnp.int32)

    def process(xi, out):
        c_gt = jnp.int32(0)
        prefix = jnp.int32(0)
        hb = jnp.int32(0)
        for p in range(4):
            shift = 24 - 8 * p
            pmask = (1 << (8 * p)) - 1

            if p == 0:
                @plsc.parallel_loop(0, 256, unroll=8)
                def _(i):
                    hist0[pl.ds(i * _L, _L)] = zeros16
            else:
                for c in range(_H * _L):
                    hist[pl.ds(c * _L, _L)] = zeros16

            @plsc.parallel_loop(0, _NV, unroll=_U)
            def _(j, _p=p, _shift=shift, _pm=pmask, _prefix=prefix):
                sl = pl.ds(j * _L, _L)
                if _p == 0:
                    iv = xi[sl]
                    ub = jnp.where(iv < 0, ~iv, iv ^ _MIN32)
                    xi[sl] = ub
                    idx = (((ub >> _shift) & 0xFF) * _L) + iota
                    plsc.addupdate_scatter(hist0, [idx], ones)
                else:
                    ub = xi[sl]
                    binv = ((ub >> _shift) & 0xFF) + (j % _H) * 256
                    sel = ((ub >> (_shift + 8)) & _pm) == _prefix
                    plsc.addupdate_scatter(hist, [binv], ones, mask=sel)

            if p == 0:
                def cbody0(c, csv):
                    vs = [hist0[pl.ds(c * 256 + t * _L, _L)]
                          for t in range(_L)]
                    while len(vs) > 1:
                        vs = [a + b for a, b in zip(vs[::2], vs[1::2])]
                    return jnp.where(iota == c, jnp.sum(vs[0]), csv)

                csv = lax.fori_loop(0, _L, cbody0, zeros16)
            else:
                csv = zeros16
                for c in range(_L):
                    acc = hist[pl.ds(c * _L, _L)]
                    for h in range(1, _H):
                        acc = acc + hist[pl.ds(h * 256 + c * _L, _L)]
                    csv = jnp.where(iota == c, jnp.sum(acc), csv)
            suff_c = jnp.flip(jnp.cumsum(jnp.flip(csv)))
            cstar = jnp.max(jnp.where((c_gt + suff_c) >= _K, iota, -1))
            rnext = jnp.max(jnp.where(iota == cstar, suff_c - csv, -1))

            if p == 0:
                def bbody0(l, w):
                    s = jnp.sum(hist0[pl.ds(cstar * 256 + l * _L, _L)])
                    return jnp.where(iota == l, s, w)

                w = lax.fori_loop(0, _L, bbody0, zeros16)
            else:
                w = hist[pl.ds(cstar * _L, _L)]
                for h in range(1, _H):
                    w = w + hist[pl.ds(h * 256 + cstar * _L, _L)]
            suff_w = jnp.flip(jnp.cumsum(jnp.flip(w)))
            bsel = jnp.max(jnp.where((c_gt + rnext + suff_w) >= _K, iota, -1))
            hb = jnp.max(jnp.where(iota == bsel, w, -1))
            suffb = jnp.max(jnp.where(iota == bsel, suff_w, -1))
            c_gt = c_gt + rnext + suffb - hb
            prefix = (prefix << 8) | (cstar * _L + bsel)

        tkey = prefix ^ _MIN32

        @plsc.parallel_loop(0, _NV, unroll=_U)
        def _(j):
            sl = pl.ds(j * _L, _L)
            ub = xi[sl]
            ks = ub ^ _MIN32
            iv = jnp.where(ks < 0, ~ub, ks)
            out[sl] = jnp.where(ks >= tkey, iv, ninfv)

        quota = jnp.int32(_K) - c_gt

        @pl.when(c_gt + hb > _K)
        def _():
            def fbody(j, cnt):
                sl = pl.ds(j * _L, _L)
                eqm = xi[sl] == prefix
                incl = jnp.cumsum(eqm.astype(jnp.int32))
                kill = eqm & ((cnt + incl) > quota)
                out[sl] = jnp.where(kill, ninfv, out[sl])
                return cnt + plsc.all_reduce_population_count(eqm)

            lax.fori_loop(0, _NV, fbody, jnp.zeros((_L,), jnp.int32))

    row = wid * _RPW
    bits_hbm = scores_hbm.at[...].bitcast(jnp.int32)
    obits_hbm = out_hbm.at[...].bitcast(jnp.int32)
    cp_ia = pltpu.async_copy(bits_hbm.at[row], xa, s_ia)
    cp_ib = pltpu.async_copy(bits_hbm.at[row + 1], xb, s_ib)
    cp_ia.wait()
    process(xa, oa)
    cp_oa = pltpu.async_copy(oa, obits_hbm.at[row], s_oa)
    cp_ib.wait()
    process(xb, ob)
    cp_ob = pltpu.async_copy(ob, obits_hbm.at[row + 1], s_ob)
    cp_oa.wait()
    cp_ob.wait()


def kernel(scores, k):
    del k
    return _sc_topk_mask(scores)
